# baseline (speedup 1.0000x reference)
import jax
import jax.numpy as jnp
from jax import lax
from jax.experimental import pallas as pl
from jax.experimental.pallas import tpu as pltpu
from jax.experimental.pallas import tpu_sc as plsc

B = 16384
D = 32
L = 16
NC = 2
NS = 16
NW = NC * NS
BPW = B // NW
NCHUNK = 4
CH = BPW // NCHUNK
NK = BPW // L

NUM_GENDER = 2
NUM_AGE = 7
NUM_OCC = 21


def _body(gidx, aidx, oidx, zidx, Wg, Wa, Wo, Wz, out,
          gi_v, ai_v, oi_v, zi_v, g_v, a_v, o_v, z_v, gt_v, at_v, ot_v,
          isem, gsem):
    wid = lax.axis_index("s") * NC + lax.axis_index("c")
    base = wid * BPW

    idx_copies = [
        pltpu.async_copy(zidx.at[pl.ds(base, BPW)], zi_v, isem),
        pltpu.async_copy(gidx.at[pl.ds(base, BPW)], gi_v, isem),
        pltpu.async_copy(aidx.at[pl.ds(base, BPW)], ai_v, isem),
        pltpu.async_copy(oidx.at[pl.ds(base, BPW)], oi_v, isem),
        pltpu.async_copy(Wg, gt_v, isem),
        pltpu.async_copy(Wa, at_v, isem),
        pltpu.async_copy(Wo, ot_v, isem),
    ]
    for c in idx_copies:
        c.wait()

    area_copies = [
        pltpu.async_copy(Wz.at[zi_v.at[pl.ds(j * CH, CH)]],
                         z_v.at[pl.ds(j * CH, CH)], gsem)
        for j in range(NCHUNK)
    ]

    iota = lax.iota(jnp.int32, L)

    def chunk(k, _):
        rowvec = iota + k * L
        for idx_v, tab_v, dst_v in ((gi_v, gt_v, g_v), (ai_v, at_v, a_v),
                                    (oi_v, ot_v, o_v)):
            idxvec = idx_v[pl.dslice(k * L, L)]
            for c in range(D):
                colvec = jnp.full((L,), c, jnp.int32)
                vals = plsc.load_gather(tab_v, [idxvec, colvec])
                plsc.store_scatter(dst_v, [rowvec, colvec], vals)
        return 0

    lax.fori_loop(0, NK, chunk, 0)

    for c in area_copies:
        c.wait()

    pltpu.sync_copy(g_v, out.at[pl.ds(base, BPW), pl.ds(0 * D, D)])
    pltpu.sync_copy(a_v, out.at[pl.ds(base, BPW), pl.ds(1 * D, D)])
    pltpu.sync_copy(o_v, out.at[pl.ds(base, BPW), pl.ds(2 * D, D)])
    pltpu.sync_copy(z_v, out.at[pl.ds(base, BPW), pl.ds(3 * D, D)])


@jax.jit
def _lookup_concat(gidx, aidx, oidx, zidx, Wg, Wa, Wo, Wz):
    mesh = plsc.VectorSubcoreMesh(core_axis_name="c", subcore_axis_name="s",
                                  num_cores=NC, num_subcores=NS)
    f = pl.kernel(
        _body, mesh=mesh,
        out_type=jax.ShapeDtypeStruct((B, 4 * D), jnp.float32),
        scratch_types=[
            pltpu.VMEM((BPW,), jnp.int32),
            pltpu.VMEM((BPW,), jnp.int32),
            pltpu.VMEM((BPW,), jnp.int32),
            pltpu.VMEM((BPW,), jnp.int32),
            pltpu.VMEM((BPW, D), jnp.float32),
            pltpu.VMEM((BPW, D), jnp.float32),
            pltpu.VMEM((BPW, D), jnp.float32),
            pltpu.VMEM((BPW, D), jnp.float32),
            pltpu.VMEM((NUM_GENDER + 1, D), jnp.float32),
            pltpu.VMEM((NUM_AGE + 1, D), jnp.float32),
            pltpu.VMEM((NUM_OCC + 1, D), jnp.float32),
            pltpu.SemaphoreType.DMA,
            pltpu.SemaphoreType.DMA,
        ],
        compiler_params=pltpu.CompilerParams(use_tc_tiling_on_sc=False, needs_layout_passes=False),
    )
    return f(gidx, aidx, oidx, zidx, Wg, Wa, Wo, Wz)


def kernel(gender_idx, age_idx, occupation_idx, area_idx,
           W_gender, W_age, W_occ, W_area):
    return _lookup_concat(
        gender_idx.astype(jnp.int32),
        age_idx.astype(jnp.int32),
        occupation_idx.astype(jnp.int32),
        area_idx.astype(jnp.int32),
        W_gender, W_age, W_occ, W_area)


# speedup vs baseline: 1.5263x; 1.5263x over previous
import jax
import jax.numpy as jnp
from jax import lax
from jax.experimental import pallas as pl
from jax.experimental.pallas import tpu as pltpu
from jax.experimental.pallas import tpu_sc as plsc

B = 16384
D = 32
L = 16
NC = 2
NS = 16
NW = NC * NS
BPW = B // NW
NCHUNK = 4
CH = BPW // NCHUNK
NK = BPW // L

NUM_GENDER = 2
NUM_AGE = 7
NUM_OCC = 21


def _body(gidx, aidx, oidx, zidx, Wg, Wa, Wo, Wz, out,
          gi_v, ai_v, oi_v, zi_v, g_v, a_v, o_v, z_v, gt_v, at_v, ot_v,
          isem, gsem):
    wid = lax.axis_index("s") * NC + lax.axis_index("c")
    base = wid * BPW

    idx_copies = [
        pltpu.async_copy(zidx.at[pl.ds(base, BPW)], zi_v, isem),
        pltpu.async_copy(gidx.at[pl.ds(base, BPW)], gi_v, isem),
        pltpu.async_copy(aidx.at[pl.ds(base, BPW)], ai_v, isem),
        pltpu.async_copy(oidx.at[pl.ds(base, BPW)], oi_v, isem),
        pltpu.async_copy(Wg, gt_v, isem),
        pltpu.async_copy(Wa, at_v, isem),
        pltpu.async_copy(Wo, ot_v, isem),
    ]
    for c in idx_copies:
        c.wait()

    area_copies = [
        pltpu.async_copy(Wz.at[zi_v.at[pl.ds(j * CH, CH)]],
                         z_v.at[pl.ds(j * CH, CH)], gsem)
        for j in range(NCHUNK)
    ]

    def row_block(rb, _):
        for idx_v, tab_v, dst_v in ((gi_v, gt_v, g_v), (ai_v, at_v, a_v),
                                    (oi_v, ot_v, o_v)):
            idxvec = idx_v[pl.ds(rb * L, L)]
            for u in range(L):
                i = idxvec[u]
                r = rb * L + u
                for h in range(D // L):
                    dst_v[r, pl.ds(h * L, L)] = tab_v[i, pl.ds(h * L, L)]
        return 0

    lax.fori_loop(0, NK, row_block, 0)

    for c in area_copies:
        c.wait()

    pltpu.sync_copy(g_v, out.at[pl.ds(base, BPW), pl.ds(0 * D, D)])
    pltpu.sync_copy(a_v, out.at[pl.ds(base, BPW), pl.ds(1 * D, D)])
    pltpu.sync_copy(o_v, out.at[pl.ds(base, BPW), pl.ds(2 * D, D)])
    pltpu.sync_copy(z_v, out.at[pl.ds(base, BPW), pl.ds(3 * D, D)])


@jax.jit
def _lookup_concat(gidx, aidx, oidx, zidx, Wg, Wa, Wo, Wz):
    mesh = plsc.VectorSubcoreMesh(core_axis_name="c", subcore_axis_name="s",
                                  num_cores=NC, num_subcores=NS)
    f = pl.kernel(
        _body, mesh=mesh,
        out_type=jax.ShapeDtypeStruct((B, 4 * D), jnp.float32),
        scratch_types=[
            pltpu.VMEM((BPW,), jnp.int32),
            pltpu.VMEM((BPW,), jnp.int32),
            pltpu.VMEM((BPW,), jnp.int32),
            pltpu.VMEM((BPW,), jnp.int32),
            pltpu.VMEM((BPW, D), jnp.float32),
            pltpu.VMEM((BPW, D), jnp.float32),
            pltpu.VMEM((BPW, D), jnp.float32),
            pltpu.VMEM((BPW, D), jnp.float32),
            pltpu.VMEM((NUM_GENDER + 1, D), jnp.float32),
            pltpu.VMEM((NUM_AGE + 1, D), jnp.float32),
            pltpu.VMEM((NUM_OCC + 1, D), jnp.float32),
            pltpu.SemaphoreType.DMA,
            pltpu.SemaphoreType.DMA,
        ],
        compiler_params=pltpu.CompilerParams(use_tc_tiling_on_sc=False, needs_layout_passes=False),
    )
    return f(gidx, aidx, oidx, zidx, Wg, Wa, Wo, Wz)


def kernel(gender_idx, age_idx, occupation_idx, area_idx,
           W_gender, W_age, W_occ, W_area):
    return _lookup_concat(
        gender_idx.astype(jnp.int32),
        age_idx.astype(jnp.int32),
        occupation_idx.astype(jnp.int32),
        area_idx.astype(jnp.int32),
        W_gender, W_age, W_occ, W_area)
